# baseline (speedup 1.0000x reference)
import functools

import jax
import jax.numpy as jnp
from jax import lax
from jax.experimental import pallas as pl
from jax.experimental.pallas import tpu as pltpu
from jax.experimental.pallas import tpu_sc as plsc

_F = 26
_V = 100000
_D = 64
_B = 4096
_EPS = 1e-5

_NW = 32
_BPW = _B * _F // _NW
_NBLK = _BPW // 128

_BT = 256
_NT = _B // _BT


def _sc_gather(table128, gid_fm):
    mesh = plsc.VectorSubcoreMesh(core_axis_name="c", subcore_axis_name="s")

    @functools.partial(
        pl.kernel,
        mesh=mesh,
        out_type=jax.ShapeDtypeStruct((_B * _F, 128), jnp.float32),
        scratch_types=[
            pltpu.VMEM((_BPW,), jnp.int32),
            pltpu.VMEM((128, 128), jnp.float32),
            pltpu.VMEM((128, 128), jnp.float32),
            pltpu.SemaphoreType.DMA,
            pltpu.SemaphoreType.DMA,
        ],
    )
    def gather_k(table_hbm, gid_hbm, out_hbm, idx_v, buf0, buf1, sem0, sem1):
        wid = lax.axis_index("s") * 2 + lax.axis_index("c")
        base = wid * _BPW
        pltpu.sync_copy(gid_hbm.at[wid], idx_v)

        @pl.loop(0, _NBLK // 2)
        def _(jj):
            j0 = jj * 2
            j1 = j0 + 1
            c0 = pltpu.async_copy(
                table_hbm.at[idx_v.at[pl.ds(j0 * 128, 128)]], buf0, sem0)
            c1 = pltpu.async_copy(
                table_hbm.at[idx_v.at[pl.ds(j1 * 128, 128)]], buf1, sem1)
            c0.wait()
            pltpu.sync_copy(buf0, out_hbm.at[pl.ds(base + j0 * 128, 128)])
            c1.wait()
            pltpu.sync_copy(buf1, out_hbm.at[pl.ds(base + j1 * 128, 128)])

    return gather_k(table128, gid_fm)


def _stats_of(u):
    return (jnp.sum(u, axis=0, keepdims=True),
            jnp.sum(u * u, axis=0, keepdims=True))


def _fold(st_ref, g_ref, be_ref):
    mu = st_ref[0:1, :] * (1.0 / _B)
    var = st_ref[1:2, :] * (1.0 / _B) - mu * mu
    s = g_ref[...] * lax.rsqrt(var + _EPS)
    t = be_ref[...] - mu * s
    return s, t


def _layer_first(dense, w, b):
    cin, cout = w.shape

    def body(x_ref, w_ref, b_ref, o_ref, st_ref):
        u = jnp.maximum(
            jnp.dot(x_ref[...], w_ref[...],
                    preferred_element_type=jnp.float32) + b_ref[...], 0.0)
        o_ref[...] = u
        cs, sq = _stats_of(u)

        @pl.when(pl.program_id(0) == 0)
        def _():
            st_ref[...] = jnp.zeros_like(st_ref)
        st_ref[0:1, :] += cs
        st_ref[1:2, :] += sq

    return pl.pallas_call(
        body,
        grid=(_NT,),
        in_specs=[
            pl.BlockSpec((_BT, cin), lambda i: (i, 0)),
            pl.BlockSpec((cin, cout), lambda i: (0, 0)),
            pl.BlockSpec((1, cout), lambda i: (0, 0)),
        ],
        out_specs=[
            pl.BlockSpec((_BT, cout), lambda i: (i, 0)),
            pl.BlockSpec((8, cout), lambda i: (0, 0)),
        ],
        out_shape=[
            jax.ShapeDtypeStruct((_B, cout), jnp.float32),
            jax.ShapeDtypeStruct((8, cout), jnp.float32),
        ],
    )(dense, w, b)


def _layer_bn(u_in, st_in, g, be, w, b):
    cin, cout = w.shape

    def body(x_ref, st_in_ref, g_ref, be_ref, w_ref, b_ref, o_ref, st_ref):
        s, t = _fold(st_in_ref, g_ref, be_ref)
        xn = x_ref[...] * s + t
        u = jnp.maximum(
            jnp.dot(xn, w_ref[...],
                    preferred_element_type=jnp.float32) + b_ref[...], 0.0)
        o_ref[...] = u
        cs, sq = _stats_of(u)

        @pl.when(pl.program_id(0) == 0)
        def _():
            st_ref[...] = jnp.zeros_like(st_ref)
        st_ref[0:1, :] += cs
        st_ref[1:2, :] += sq

    return pl.pallas_call(
        body,
        grid=(_NT,),
        in_specs=[
            pl.BlockSpec((_BT, cin), lambda i: (i, 0)),
            pl.BlockSpec((8, cin), lambda i: (0, 0)),
            pl.BlockSpec((1, cin), lambda i: (0, 0)),
            pl.BlockSpec((1, cin), lambda i: (0, 0)),
            pl.BlockSpec((cin, cout), lambda i: (0, 0)),
            pl.BlockSpec((1, cout), lambda i: (0, 0)),
        ],
        out_specs=[
            pl.BlockSpec((_BT, cout), lambda i: (i, 0)),
            pl.BlockSpec((8, cout), lambda i: (0, 0)),
        ],
        out_shape=[
            jax.ShapeDtypeStruct((_B, cout), jnp.float32),
            jax.ShapeDtypeStruct((8, cout), jnp.float32),
        ],
    )(u_in, st_in, g, be, w, b)


def _interact(embs3, u2, st2, g2, be2, msym, wzx, wx, b0):
    cout = wx.shape[1]

    def body(e_ref, x_ref, st_ref, g_ref, be_ref, m_ref, wzx_ref, wx_ref,
             b_ref, o_ref, sto_ref):
        s, t = _fold(st_ref, g_ref, be_ref)
        xx = x_ref[...] * s + t
        E = e_ref[:, :, :_D]
        Zee = lax.dot_general(E, E, (((2,), (2,)), ((1,), (1,))))
        zex = jnp.sum(E * xx[None, :, :], axis=2)
        acc = (jnp.dot(xx, wx_ref[...], preferred_element_type=jnp.float32)
               + lax.dot_general(zex, wzx_ref[...],
                                 (((0,), (0,)), ((), ())),
                                 preferred_element_type=jnp.float32)
               + b_ref[...])
        for i in range(_F):
            acc = acc + jnp.dot(Zee[:, i, :], m_ref[i],
                                preferred_element_type=jnp.float32)
        u = jnp.maximum(acc, 0.0)
        o_ref[...] = u
        cs, sq = _stats_of(u)

        @pl.when(pl.program_id(0) == 0)
        def _():
            sto_ref[...] = jnp.zeros_like(sto_ref)
        sto_ref[0:1, :] += cs
        sto_ref[1:2, :] += sq

    return pl.pallas_call(
        body,
        grid=(_NT,),
        in_specs=[
            pl.BlockSpec((_F, _BT, 128), lambda i: (0, i, 0)),
            pl.BlockSpec((_BT, _D), lambda i: (i, 0)),
            pl.BlockSpec((8, _D), lambda i: (0, 0)),
            pl.BlockSpec((1, _D), lambda i: (0, 0)),
            pl.BlockSpec((1, _D), lambda i: (0, 0)),
            pl.BlockSpec((_F, _F, cout), lambda i: (0, 0, 0)),
            pl.BlockSpec((_F, cout), lambda i: (0, 0)),
            pl.BlockSpec((_D, cout), lambda i: (0, 0)),
            pl.BlockSpec((1, cout), lambda i: (0, 0)),
        ],
        out_specs=[
            pl.BlockSpec((_BT, cout), lambda i: (i, 0)),
            pl.BlockSpec((8, cout), lambda i: (0, 0)),
        ],
        out_shape=[
            jax.ShapeDtypeStruct((_B, cout), jnp.float32),
            jax.ShapeDtypeStruct((8, cout), jnp.float32),
        ],
    )(embs3, u2, st2, g2, be2, msym, wzx, wx, b0)


def _final(u, st, g, be):
    def body(x_ref, st_ref, g_ref, be_ref, o_ref):
        s, t = _fold(st_ref, g_ref, be_ref)
        o_ref[...] = jax.nn.sigmoid(x_ref[...] * s + t)

    return pl.pallas_call(
        body,
        grid=(_NT,),
        in_specs=[
            pl.BlockSpec((_BT, 1), lambda i: (i, 0)),
            pl.BlockSpec((8, 1), lambda i: (0, 0)),
            pl.BlockSpec((1, 1), lambda i: (0, 0)),
            pl.BlockSpec((1, 1), lambda i: (0, 0)),
        ],
        out_specs=pl.BlockSpec((_BT, 1), lambda i: (i, 0)),
        out_shape=jax.ShapeDtypeStruct((_B, 1), jnp.float32),
    )(u, st, g, be)


def kernel(sparse_inputs, dense_inputs, emb, params):
    p = params

    table128 = jnp.concatenate(
        [jnp.pad(emb[f], ((0, 0), (0, 128 - _D))) for f in range(_F)], axis=0)
    gid_fm = (sparse_inputs.T.astype(jnp.int32)
              + (jnp.arange(_F, dtype=jnp.int32) * _V)[:, None]
              ).reshape(_NW, _BPW)
    rows = _sc_gather(table128, gid_fm)
    embs3 = rows.reshape(_F, _B, 128)

    def wt(i, pre):
        return p[f'{pre}{i}_W'].T, p[f'{pre}{i}_b'].reshape(1, -1)

    w0, b0 = wt(0, 'bot')
    w1, b1 = wt(1, 'bot')
    w2, b2 = wt(2, 'bot')
    tw0 = p['top0_W']
    wx = tw0[:, :_D].T
    wzt = tw0[:, _D:].T
    iu0, iu1 = jnp.triu_indices(_F + 1, k=1)
    m_full = jnp.zeros((_F + 1, _F + 1, tw0.shape[0]), jnp.float32)
    m_full = m_full.at[iu0, iu1].set(wzt)
    msym = m_full[:_F, :_F, :]
    wzx = m_full[:_F, _F, :]
    tb0 = p['top0_b'].reshape(1, -1)
    w4, b4 = wt(1, 'top')
    w5, b5 = wt(2, 'top')

    def bn(i, pre):
        return (p[f'{pre}{i}_g'].reshape(1, -1),
                p[f'{pre}{i}_beta'].reshape(1, -1))

    g0, be0 = bn(0, 'bot')
    g1, be1 = bn(1, 'bot')
    g2, be2 = bn(2, 'bot')
    g3, be3 = bn(0, 'top')
    g4, be4 = bn(1, 'top')
    g5, be5 = bn(2, 'top')

    u0, st0 = _layer_first(dense_inputs, w0, b0)
    u1, st1 = _layer_bn(u0, st0, g0, be0, w1, b1)
    u2, st2 = _layer_bn(u1, st1, g1, be1, w2, b2)
    u3, st3 = _interact(embs3, u2, st2, g2, be2, msym, wzx, wx, tb0)
    u4, st4 = _layer_bn(u3, st3, g3, be3, w4, b4)
    u5, st5 = _layer_bn(u4, st4, g4, be4, w5, b5)
    return _final(u5, st5, g5, be5)


# speedup vs baseline: 1.5827x; 1.5827x over previous
import functools

import jax
import jax.numpy as jnp
from jax import lax
from jax.experimental import pallas as pl
from jax.experimental.pallas import tpu as pltpu
from jax.experimental.pallas import tpu_sc as plsc

_F = 26
_V = 100000
_D = 64
_B = 4096
_EPS = 1e-5

_NW = 32
_BPW = _B * _F // _NW
_NBLK = _BPW // 128

_BT = 256
_NT = _B // _BT


def _sc_gather(table128, gid_fm):
    mesh = plsc.VectorSubcoreMesh(core_axis_name="c", subcore_axis_name="s")

    @functools.partial(
        pl.kernel,
        mesh=mesh,
        out_type=jax.ShapeDtypeStruct((_B * _F, 128), jnp.float32),
        scratch_types=[
            pltpu.VMEM((_BPW,), jnp.int32),
            pltpu.VMEM((128, 128), jnp.float32),
            pltpu.VMEM((128, 128), jnp.float32),
            pltpu.SemaphoreType.DMA,
            pltpu.SemaphoreType.DMA,
        ],
    )
    def gather_k(table_hbm, gid_hbm, out_hbm, idx_v, buf0, buf1, sem0, sem1):
        wid = lax.axis_index("s") * 2 + lax.axis_index("c")
        base = wid * _BPW
        pltpu.sync_copy(gid_hbm.at[wid], idx_v)

        @pl.loop(0, _NBLK // 2)
        def _(jj):
            j0 = jj * 2
            j1 = j0 + 1
            c0 = pltpu.async_copy(
                table_hbm.at[idx_v.at[pl.ds(j0 * 128, 128)]], buf0, sem0)
            c1 = pltpu.async_copy(
                table_hbm.at[idx_v.at[pl.ds(j1 * 128, 128)]], buf1, sem1)
            c0.wait()
            pltpu.sync_copy(buf0, out_hbm.at[pl.ds(base + j0 * 128, 128)])
            c1.wait()
            pltpu.sync_copy(buf1, out_hbm.at[pl.ds(base + j1 * 128, 128)])

    return gather_k(table128, gid_fm)


def _stats_of(u):
    return (jnp.sum(u, axis=0, keepdims=True),
            jnp.sum(u * u, axis=0, keepdims=True))


def _fold(st_ref, g_ref, be_ref):
    mu = st_ref[0:1, :] * (1.0 / _B)
    var = st_ref[1:2, :] * (1.0 / _B) - mu * mu
    s = g_ref[...] * lax.rsqrt(var + _EPS)
    t = be_ref[...] - mu * s
    return s, t


def _layer_first(dense, w, b):
    cin, cout = w.shape

    def body(x_ref, w_ref, b_ref, o_ref, st_ref):
        u = jnp.maximum(
            jnp.dot(x_ref[...], w_ref[...],
                    preferred_element_type=jnp.float32) + b_ref[...], 0.0)
        o_ref[...] = u
        cs, sq = _stats_of(u)

        @pl.when(pl.program_id(0) == 0)
        def _():
            st_ref[...] = jnp.zeros_like(st_ref)
        st_ref[0:1, :] += cs
        st_ref[1:2, :] += sq

    return pl.pallas_call(
        body,
        grid=(_NT,),
        in_specs=[
            pl.BlockSpec((_BT, cin), lambda i: (i, 0)),
            pl.BlockSpec((cin, cout), lambda i: (0, 0)),
            pl.BlockSpec((1, cout), lambda i: (0, 0)),
        ],
        out_specs=[
            pl.BlockSpec((_BT, cout), lambda i: (i, 0)),
            pl.BlockSpec((8, cout), lambda i: (0, 0)),
        ],
        out_shape=[
            jax.ShapeDtypeStruct((_B, cout), jnp.float32),
            jax.ShapeDtypeStruct((8, cout), jnp.float32),
        ],
    )(dense, w, b)


def _layer_bn(u_in, st_in, g, be, w, b):
    cin, cout = w.shape

    def body(x_ref, st_in_ref, g_ref, be_ref, w_ref, b_ref, o_ref, st_ref):
        s, t = _fold(st_in_ref, g_ref, be_ref)
        xn = x_ref[...] * s + t
        u = jnp.maximum(
            jnp.dot(xn, w_ref[...],
                    preferred_element_type=jnp.float32) + b_ref[...], 0.0)
        o_ref[...] = u
        cs, sq = _stats_of(u)

        @pl.when(pl.program_id(0) == 0)
        def _():
            st_ref[...] = jnp.zeros_like(st_ref)
        st_ref[0:1, :] += cs
        st_ref[1:2, :] += sq

    return pl.pallas_call(
        body,
        grid=(_NT,),
        in_specs=[
            pl.BlockSpec((_BT, cin), lambda i: (i, 0)),
            pl.BlockSpec((8, cin), lambda i: (0, 0)),
            pl.BlockSpec((1, cin), lambda i: (0, 0)),
            pl.BlockSpec((1, cin), lambda i: (0, 0)),
            pl.BlockSpec((cin, cout), lambda i: (0, 0)),
            pl.BlockSpec((1, cout), lambda i: (0, 0)),
        ],
        out_specs=[
            pl.BlockSpec((_BT, cout), lambda i: (i, 0)),
            pl.BlockSpec((8, cout), lambda i: (0, 0)),
        ],
        out_shape=[
            jax.ShapeDtypeStruct((_B, cout), jnp.float32),
            jax.ShapeDtypeStruct((8, cout), jnp.float32),
        ],
    )(u_in, st_in, g, be, w, b)


def _interact(embs3, u2, st2, g2, be2, msym, wzx, wx, b0):
    cout = wx.shape[1]

    def body(e_ref, x_ref, st_ref, g_ref, be_ref, m_ref, wzx_ref, wx_ref,
             b_ref, o_ref, sto_ref):
        s, t = _fold(st_ref, g_ref, be_ref)
        xx = x_ref[...] * s + t
        E = e_ref[:, :, :_D]
        Zee = lax.dot_general(E, E, (((2,), (2,)), ((0,), (0,))))
        zex = jnp.sum(E * xx[:, None, :], axis=2)
        acc = (jnp.dot(xx, wx_ref[...], preferred_element_type=jnp.float32)
               + jnp.dot(zex, wzx_ref[...], preferred_element_type=jnp.float32)
               + b_ref[...])
        for i in range(_F):
            acc = acc + jnp.dot(Zee[:, i, :], m_ref[i],
                                preferred_element_type=jnp.float32)
        u = jnp.maximum(acc, 0.0)
        o_ref[...] = u
        cs, sq = _stats_of(u)

        @pl.when(pl.program_id(0) == 0)
        def _():
            sto_ref[...] = jnp.zeros_like(sto_ref)
        sto_ref[0:1, :] += cs
        sto_ref[1:2, :] += sq

    return pl.pallas_call(
        body,
        grid=(_NT,),
        in_specs=[
            pl.BlockSpec((_BT, _F, 128), lambda i: (i, 0, 0)),
            pl.BlockSpec((_BT, _D), lambda i: (i, 0)),
            pl.BlockSpec((8, _D), lambda i: (0, 0)),
            pl.BlockSpec((1, _D), lambda i: (0, 0)),
            pl.BlockSpec((1, _D), lambda i: (0, 0)),
            pl.BlockSpec((_F, _F, cout), lambda i: (0, 0, 0)),
            pl.BlockSpec((_F, cout), lambda i: (0, 0)),
            pl.BlockSpec((_D, cout), lambda i: (0, 0)),
            pl.BlockSpec((1, cout), lambda i: (0, 0)),
        ],
        out_specs=[
            pl.BlockSpec((_BT, cout), lambda i: (i, 0)),
            pl.BlockSpec((8, cout), lambda i: (0, 0)),
        ],
        out_shape=[
            jax.ShapeDtypeStruct((_B, cout), jnp.float32),
            jax.ShapeDtypeStruct((8, cout), jnp.float32),
        ],
    )(embs3, u2, st2, g2, be2, msym, wzx, wx, b0)


def _final(u, st, g, be):
    def body(x_ref, st_ref, g_ref, be_ref, o_ref):
        s, t = _fold(st_ref, g_ref, be_ref)
        o_ref[...] = jax.nn.sigmoid(x_ref[...] * s + t)

    return pl.pallas_call(
        body,
        grid=(_NT,),
        in_specs=[
            pl.BlockSpec((_BT, 1), lambda i: (i, 0)),
            pl.BlockSpec((8, 1), lambda i: (0, 0)),
            pl.BlockSpec((1, 1), lambda i: (0, 0)),
            pl.BlockSpec((1, 1), lambda i: (0, 0)),
        ],
        out_specs=pl.BlockSpec((_BT, 1), lambda i: (i, 0)),
        out_shape=jax.ShapeDtypeStruct((_B, 1), jnp.float32),
    )(u, st, g, be)


def kernel(sparse_inputs, dense_inputs, emb, params):
    p = params

    table128 = jnp.pad(emb, ((0, 0), (0, 0), (0, 128 - _D))
                       ).reshape(_F * _V, 128)
    gid_bm = (sparse_inputs.astype(jnp.int32)
              + (jnp.arange(_F, dtype=jnp.int32) * _V)[None, :]
              ).reshape(_NW, _BPW)
    rows = _sc_gather(table128, gid_bm)
    embs3 = rows.reshape(_B, _F, 128)

    def wt(i, pre):
        return p[f'{pre}{i}_W'].T, p[f'{pre}{i}_b'].reshape(1, -1)

    w0, b0 = wt(0, 'bot')
    w1, b1 = wt(1, 'bot')
    w2, b2 = wt(2, 'bot')
    tw0 = p['top0_W']
    wx = tw0[:, :_D].T
    wzt = tw0[:, _D:].T
    iu0, iu1 = jnp.triu_indices(_F + 1, k=1)
    m_full = jnp.zeros((_F + 1, _F + 1, tw0.shape[0]), jnp.float32)
    m_full = m_full.at[iu0, iu1].set(wzt)
    msym = m_full[:_F, :_F, :]
    wzx = m_full[:_F, _F, :]
    tb0 = p['top0_b'].reshape(1, -1)
    w4, b4 = wt(1, 'top')
    w5, b5 = wt(2, 'top')

    def bn(i, pre):
        return (p[f'{pre}{i}_g'].reshape(1, -1),
                p[f'{pre}{i}_beta'].reshape(1, -1))

    g0, be0 = bn(0, 'bot')
    g1, be1 = bn(1, 'bot')
    g2, be2 = bn(2, 'bot')
    g3, be3 = bn(0, 'top')
    g4, be4 = bn(1, 'top')
    g5, be5 = bn(2, 'top')

    u0, st0 = _layer_first(dense_inputs, w0, b0)
    u1, st1 = _layer_bn(u0, st0, g0, be0, w1, b1)
    u2, st2 = _layer_bn(u1, st1, g1, be1, w2, b2)
    u3, st3 = _interact(embs3, u2, st2, g2, be2, msym, wzx, wx, tb0)
    u4, st4 = _layer_bn(u3, st3, g3, be3, w4, b4)
    u5, st5 = _layer_bn(u4, st4, g4, be4, w5, b5)
    return _final(u5, st5, g5, be5)
